# Initial kernel scaffold; baseline (speedup 1.0000x reference)
import jax
import jax.numpy as jnp
from jax.experimental import pallas as pl


def kernel(x, w1b, b1c, w2b, b2c, fw1p, fb1c, fw2p, fb2c):
    raise NotImplementedError("write your pallas kernel here")


# speedup vs baseline: 1.3582x; 1.3582x over previous
import jax
import jax.numpy as jnp
from jax.experimental import pallas as pl
from jax.experimental.pallas import tpu as pltpu

_BM = 256
_IN_FEATS = 392
_C1_COLS = 9 * 512


def _net_kernel(x_ref, w1t_ref, b1r_ref, w2t_ref, b2r_ref,
                fw1t_ref, fb1r_ref, fw2t_ref, fb2_ref, out_ref):
    f32 = jnp.float32
    xb = x_ref[...].astype(jnp.bfloat16)

    p1 = jnp.dot(xb, w1t_ref[:, 0:512], preferred_element_type=f32)
    for t in range(1, 9):
        c = jnp.dot(xb, w1t_ref[:, t * 512:(t + 1) * 512],
                    preferred_element_type=f32)
        p1 = jnp.maximum(p1, c)
    p1 = jnp.maximum(p1 + b1r_ref[...], 0.0)

    c2 = jnp.dot(p1.astype(jnp.bfloat16), w2t_ref[...],
                 preferred_element_type=f32) + b2r_ref[...]
    p2 = jnp.maximum(jnp.maximum(c2[:, 0:64], c2[:, 64:128]),
                     jnp.maximum(c2[:, 128:192], c2[:, 192:256]))
    p2 = jnp.maximum(p2, 0.0)

    h = jnp.dot(p2.astype(jnp.bfloat16), fw1t_ref[...],
                preferred_element_type=f32) + fb1r_ref[...]
    h = jnp.maximum(h, 0.0)

    o8 = jnp.dot(h.astype(jnp.bfloat16), fw2t_ref[...],
                 preferred_element_type=f32)
    o = o8[:, 0:1] + fb2_ref[...]
    out_ref[...] = jnp.tanh(jnp.maximum(o, 0.0)).astype(out_ref.dtype)


def kernel(x, w1b, b1c, w2b, b2c, fw1p, fb1c, fw2p, fb2c):
    n = x.shape[0]
    bm = _BM
    n_pad = ((n + bm - 1) // bm) * bm
    xf = x.reshape(n, _IN_FEATS)
    if n_pad != n:
        xf = jnp.pad(xf, ((0, n_pad - n), (0, 0)))

    w1t = w1b.T
    w2t = w2b.T
    fw1t = fw1p.T
    fw2t = fw2p.T
    b1r = b1c.reshape(1, 512)
    b2r = b2c.reshape(1, 256)
    fb1r = fb1c.reshape(1, 256)

    grid = (n_pad // bm,)
    in_specs = [
        pl.BlockSpec((bm, _IN_FEATS), lambda b: (b, 0)),
        pl.BlockSpec((_IN_FEATS, _C1_COLS), lambda b: (0, 0)),
        pl.BlockSpec((1, 512), lambda b: (0, 0)),
        pl.BlockSpec((512, 256), lambda b: (0, 0)),
        pl.BlockSpec((1, 256), lambda b: (0, 0)),
        pl.BlockSpec((64, 256), lambda b: (0, 0)),
        pl.BlockSpec((1, 256), lambda b: (0, 0)),
        pl.BlockSpec((256, 8), lambda b: (0, 0)),
        pl.BlockSpec((1, 1), lambda b: (0, 0)),
    ]
    out_specs = pl.BlockSpec((bm, 1), lambda b: (b, 0))

    out = pl.pallas_call(
        _net_kernel,
        out_shape=jax.ShapeDtypeStruct((n_pad, 1), jnp.float32),
        grid=grid,
        in_specs=in_specs,
        out_specs=out_specs,
        compiler_params=pltpu.CompilerParams(
            dimension_semantics=("parallel",),
            vmem_limit_bytes=64 * 1024 * 1024,
        ),
    )(xf, w1t, b1r, w2t, b2r, fw1t, fb1r, fw2t, fb2c)

    return out[:n, :].astype(x.dtype)
